# baseline (speedup 1.0000x reference)
import jax
import jax.numpy as jnp
from jax import lax
from jax.experimental import pallas as pl
from jax.experimental.pallas import tpu as pltpu
from jax.experimental.pallas import tpu_sc as plsc

NC = 2
NS = 16
NW = NC * NS
B = 320000
V = 10000
D = 128
E = 200
PER_W = B // (NW * E)
EW = B // NW
GROUPS = (E + 15) // 16
EP = GROUPS * 16
DEPTH = 2
CR = 25
CCH = V // (CR * NS)


def _body(eidx_hbm, emb_hbm, out_hbm, ci0, co0, ci1, co1, spm,
          idx_s, idx_o, rs0, ro0, rs1, ro1, ov0, ov1,
          sem0, sem1, semi0, semi1, semo0, semo1):
    cid = lax.axis_index("c")
    sid = lax.axis_index("s")
    wid = sid * NC + cid
    ebase = wid * EW
    lane = lax.iota(jnp.int32, 16)
    perms = [lane ^ m for m in (8, 4, 2, 1)]
    bufs = [(rs0, ro0, sem0), (rs1, ro1, sem1)]

    pltpu.async_copy(eidx_hbm.at[0, pl.ds(ebase, EW)], idx_s, sem0)
    pltpu.async_copy(eidx_hbm.at[1, pl.ds(ebase, EW)], idx_o, sem1)

    rbase = sid * (CR * CCH)
    cbufs = [(ci0, co0, semo0), (ci1, co1, semo1)]

    def conv_in_start(k, par):
        ci, _, _ = cbufs[par]
        pltpu.async_copy(emb_hbm.at[pl.ds(rbase + k * CR, CR), :], ci,
                         semi0 if par == 0 else semi1)

    def conv_in_wait(k, par):
        ci, _, _ = cbufs[par]
        pltpu.make_async_copy(emb_hbm.at[pl.ds(rbase + k * CR, CR), :], ci,
                              semi0 if par == 0 else semi1).wait()

    def conv_out_wait(k, par):
        _, co, semo = cbufs[par]
        pltpu.make_async_copy(co, spm.at[pl.ds(rbase + k * CR, CR), :],
                              semo).wait()

    def convert(k, par, wait_out):
        ci, co, semo = cbufs[par]
        conv_in_wait(k, par)
        if wait_out:
            conv_out_wait(k - 2, par)
        for r in range(CR):
            for j in range(D // 32):
                a = ci[r, pl.ds(j * 32, 32)]
                co[r, pl.ds(j * 16, 16)] = plsc.bitcast(a, jnp.int32)
        pltpu.async_copy(co, spm.at[pl.ds(rbase + k * CR, CR), :], semo)

    conv_in_start(0, 0)
    conv_in_start(1, 1)

    def conv_pair(p, carry):
        k0 = 2 * p
        convert(k0, 0, wait_out=True)
        conv_in_start(k0 + 2, 0)
        convert(k0 + 1, 1, wait_out=True)
        conv_in_start(k0 + 3, 1)
        return carry

    convert(0, 0, wait_out=False)
    conv_in_start(2, 0)
    convert(1, 1, wait_out=False)
    conv_in_start(3, 1)
    lax.fori_loop(1, 11, conv_pair, 0)
    convert(22, 0, wait_out=True)
    conv_in_start(24, 0)
    convert(23, 1, wait_out=True)
    convert(24, 0, wait_out=True)
    conv_out_wait(23, 1)
    conv_out_wait(24, 0)
    plsc.subcore_barrier()

    pltpu.make_async_copy(eidx_hbm.at[0, pl.ds(ebase, EW)], idx_s,
                          sem0).wait()
    pltpu.make_async_copy(eidx_hbm.at[1, pl.ds(ebase, EW)], idx_o,
                          sem1).wait()

    def start(c, k):
        rows_s, rows_o, sem = bufs[k]
        pltpu.async_copy(spm.at[idx_s.at[pl.ds(c * E, E)]],
                         rows_s.at[pl.ds(0, E)], sem)
        pltpu.async_copy(spm.at[idx_o.at[pl.ds(c * E, E)]],
                         rows_o.at[pl.ds(0, E)], sem)

    def finish(c, k):
        rows_s, rows_o, sem = bufs[k]
        pltpu.make_async_copy(spm.at[idx_s.at[pl.ds(c * E, E)]],
                              rows_s.at[pl.ds(0, E)], sem).wait()
        pltpu.make_async_copy(spm.at[idx_o.at[pl.ds(c * E, E)]],
                              rows_o.at[pl.ds(0, E)], sem).wait()

    ovs = [(ov0, semo0), (ov1, semo1)]

    def compute(c, k):
        rows_s, rows_o, _ = bufs[k]
        out_v, _ = ovs[k]

        def group(g, carry2):
            outv = jnp.zeros((16,), jnp.float32)
            for l in range(16):
                e = g * 16 + l
                acc = jnp.zeros((16,), jnp.float32)
                for j in range(4):
                    sv = plsc.bitcast(rows_s[e, pl.ds(j * 16, 16)],
                                      jnp.bfloat16)
                    ov = plsc.bitcast(rows_o[e, pl.ds(j * 16, 16)],
                                      jnp.bfloat16)
                    p = sv * ov
                    plo, phi = plsc.unpack(p, format=plsc.PackFormat.INTERLEAVED)
                    acc = acc + plo + phi
                for pm in perms:
                    acc = acc + acc.at[pm].get(mode="promise_in_bounds",
                                               unique_indices=True)
                outv = jnp.where(lane == l, acc, outv)
            out_v[pl.ds(g * 16, 16)] = outv
            return carry2

        lax.fori_loop(0, GROUPS, group, 0)

    def store_start(c, k):
        out_v, semo = ovs[k]
        pltpu.async_copy(out_v.at[pl.ds(0, E)],
                         out_hbm.at[pl.ds(ebase + c * E, E)], semo)

    def store_wait(c, k):
        out_v, semo = ovs[k]
        pltpu.make_async_copy(out_v.at[pl.ds(0, E)],
                              out_hbm.at[pl.ds(ebase + c * E, E)],
                              semo).wait()

    start(0, 0)
    start(1, 1)
    for k in range(2):
        finish(k, k)
        compute(k, k)
        store_start(k, k)
        start(k + 2, k)

    def step(i, carry):
        c0 = DEPTH * i
        for k in range(DEPTH):
            finish(c0 + k, k)
            store_wait(c0 + k - 2, k)
            compute(c0 + k, k)
            store_start(c0 + k, k)
            start(c0 + k + 2, k)
        return carry

    lax.fori_loop(1, (PER_W - 2) // DEPTH, step, 0)

    for c in (PER_W - 2, PER_W - 1):
        finish(c, c % DEPTH)
        store_wait(c - 2, c % DEPTH)
        compute(c, c % DEPTH)
        store_start(c, c % DEPTH)
    store_wait(PER_W - 2, 0)
    store_wait(PER_W - 1, 1)


def kernel(embeddings, edge_index):
    emb16 = embeddings.astype(jnp.bfloat16)
    eidx = edge_index.astype(jnp.int32)
    mesh = plsc.VectorSubcoreMesh(core_axis_name="c", subcore_axis_name="s")
    rows_t = pltpu.VMEM((EP, D // 2), jnp.int32)
    run = pl.kernel(
        _body,
        out_type=jax.ShapeDtypeStruct((B,), jnp.float32),
        mesh=mesh,
        compiler_params=pltpu.CompilerParams(needs_layout_passes=False,
                                             use_tc_tiling_on_sc=False),
        scratch_types=[
            pltpu.VMEM((CR, D), jnp.bfloat16),
            pltpu.VMEM((CR, D // 2), jnp.int32),
            pltpu.VMEM((CR, D), jnp.bfloat16),
            pltpu.VMEM((CR, D // 2), jnp.int32),
            pltpu.VMEM_SHARED((V, D // 2), jnp.int32),
            pltpu.VMEM((EW,), jnp.int32),
            pltpu.VMEM((EW,), jnp.int32),
            rows_t, rows_t, rows_t, rows_t,
            pltpu.VMEM((EP,), jnp.float32),
            pltpu.VMEM((EP,), jnp.float32),
            pltpu.SemaphoreType.DMA,
            pltpu.SemaphoreType.DMA,
            pltpu.SemaphoreType.DMA,
            pltpu.SemaphoreType.DMA,
            pltpu.SemaphoreType.DMA,
            pltpu.SemaphoreType.DMA,
        ],
    )
    return run(eidx, emb16)


# speedup vs baseline: 1.0253x; 1.0253x over previous
import jax
import jax.numpy as jnp
from jax import lax
from jax.experimental import pallas as pl
from jax.experimental.pallas import tpu as pltpu
from jax.experimental.pallas import tpu_sc as plsc

NC = 2
NS = 16
NW = NC * NS
B = 320000
V = 10000
D = 128
E = 200
PER_W = B // (NW * E)
EW = B // NW
GROUPS = (E + 15) // 16
EP = GROUPS * 16
DEPTH = 2
CR = 25
CCH = V // (CR * NS)


def _body(eidx_hbm, emb_hbm, out_hbm, ci0, co0, ci1, co1, spm,
          idx_s, idx_o, rs0, ro0, rs1, ro1, ov0, ov1,
          sem0, sem1, semi0, semi1, semo0, semo1):
    cid = lax.axis_index("c")
    sid = lax.axis_index("s")
    wid = sid * NC + cid
    ebase = wid * EW
    lane = lax.iota(jnp.int32, 16)
    perms = [lane ^ m for m in (8, 4, 2, 1)]
    bufs = [(rs0, ro0, sem0), (rs1, ro1, sem1)]

    pltpu.async_copy(eidx_hbm.at[0, pl.ds(ebase, EW)], idx_s, sem0)
    pltpu.async_copy(eidx_hbm.at[1, pl.ds(ebase, EW)], idx_o, sem1)

    rbase = sid * (CR * CCH)
    cbufs = [(ci0, co0, semo0), (ci1, co1, semo1)]

    def conv_in_start(k, par):
        ci, _, _ = cbufs[par]
        pltpu.async_copy(emb_hbm.at[pl.ds(rbase + k * CR, CR), :], ci,
                         semi0 if par == 0 else semi1)

    def conv_in_wait(k, par):
        ci, _, _ = cbufs[par]
        pltpu.make_async_copy(emb_hbm.at[pl.ds(rbase + k * CR, CR), :], ci,
                              semi0 if par == 0 else semi1).wait()

    def conv_out_wait(k, par):
        _, co, semo = cbufs[par]
        pltpu.make_async_copy(co, spm.at[pl.ds(rbase + k * CR, CR), :],
                              semo).wait()

    def convert(k, par, wait_out):
        ci, co, semo = cbufs[par]
        conv_in_wait(k, par)
        if wait_out:
            conv_out_wait(k - 2, par)
        for r in range(CR):
            for j in range(D // 32):
                a = ci[r, pl.ds(j * 32, 16)]
                b = ci[r, pl.ds(j * 32 + 16, 16)]
                packed = plsc.pack(a, b, format=plsc.PackFormat.INTERLEAVED)
                co[r, pl.ds(j * 16, 16)] = plsc.bitcast(packed, jnp.int32)
        pltpu.async_copy(co, spm.at[pl.ds(rbase + k * CR, CR), :], semo)

    conv_in_start(0, 0)
    conv_in_start(1, 1)

    def conv_pair(p, carry):
        k0 = 2 * p
        convert(k0, 0, wait_out=True)
        conv_in_start(k0 + 2, 0)
        convert(k0 + 1, 1, wait_out=True)
        conv_in_start(k0 + 3, 1)
        return carry

    convert(0, 0, wait_out=False)
    conv_in_start(2, 0)
    convert(1, 1, wait_out=False)
    conv_in_start(3, 1)
    lax.fori_loop(1, 11, conv_pair, 0)
    convert(22, 0, wait_out=True)
    conv_in_start(24, 0)
    convert(23, 1, wait_out=True)
    convert(24, 0, wait_out=True)
    conv_out_wait(23, 1)
    conv_out_wait(24, 0)
    plsc.subcore_barrier()

    pltpu.make_async_copy(eidx_hbm.at[0, pl.ds(ebase, EW)], idx_s,
                          sem0).wait()
    pltpu.make_async_copy(eidx_hbm.at[1, pl.ds(ebase, EW)], idx_o,
                          sem1).wait()

    def start(c, k):
        rows_s, rows_o, sem = bufs[k]
        pltpu.async_copy(spm.at[idx_s.at[pl.ds(c * E, E)]],
                         rows_s.at[pl.ds(0, E)], sem)
        pltpu.async_copy(spm.at[idx_o.at[pl.ds(c * E, E)]],
                         rows_o.at[pl.ds(0, E)], sem)

    def finish(c, k):
        rows_s, rows_o, sem = bufs[k]
        pltpu.make_async_copy(spm.at[idx_s.at[pl.ds(c * E, E)]],
                              rows_s.at[pl.ds(0, E)], sem).wait()
        pltpu.make_async_copy(spm.at[idx_o.at[pl.ds(c * E, E)]],
                              rows_o.at[pl.ds(0, E)], sem).wait()

    ovs = [(ov0, semo0), (ov1, semo1)]

    def compute(c, k):
        rows_s, rows_o, _ = bufs[k]
        out_v, _ = ovs[k]

        def group(g, carry2):
            outv = jnp.zeros((16,), jnp.float32)
            for l in range(16):
                e = g * 16 + l
                acc = jnp.zeros((16,), jnp.float32)
                for j in range(4):
                    sv = plsc.bitcast(rows_s[e, pl.ds(j * 16, 16)],
                                      jnp.bfloat16)
                    ov = plsc.bitcast(rows_o[e, pl.ds(j * 16, 16)],
                                      jnp.bfloat16)
                    p = sv * ov
                    plo, phi = plsc.unpack(p, format=plsc.PackFormat.INTERLEAVED)
                    acc = acc + plo + phi
                for pm in perms:
                    acc = acc + acc.at[pm].get(mode="promise_in_bounds",
                                               unique_indices=True)
                outv = jnp.where(lane == l, acc, outv)
            out_v[pl.ds(g * 16, 16)] = outv
            return carry2

        lax.fori_loop(0, GROUPS, group, 0)

    def store_start(c, k):
        out_v, semo = ovs[k]
        pltpu.async_copy(out_v.at[pl.ds(0, E)],
                         out_hbm.at[pl.ds(ebase + c * E, E)], semo)

    def store_wait(c, k):
        out_v, semo = ovs[k]
        pltpu.make_async_copy(out_v.at[pl.ds(0, E)],
                              out_hbm.at[pl.ds(ebase + c * E, E)],
                              semo).wait()

    start(0, 0)
    start(1, 1)
    for k in range(2):
        finish(k, k)
        compute(k, k)
        store_start(k, k)
        start(k + 2, k)

    def step(i, carry):
        c0 = DEPTH * i
        for k in range(DEPTH):
            finish(c0 + k, k)
            store_wait(c0 + k - 2, k)
            compute(c0 + k, k)
            store_start(c0 + k, k)
            start(c0 + k + 2, k)
        return carry

    lax.fori_loop(1, (PER_W - 2) // DEPTH, step, 0)

    for c in (PER_W - 2, PER_W - 1):
        finish(c, c % DEPTH)
        store_wait(c - 2, c % DEPTH)
        compute(c, c % DEPTH)
        store_start(c, c % DEPTH)
    store_wait(PER_W - 2, 0)
    store_wait(PER_W - 1, 1)


def kernel(embeddings, edge_index):
    eidx = edge_index.astype(jnp.int32)
    mesh = plsc.VectorSubcoreMesh(core_axis_name="c", subcore_axis_name="s")
    rows_t = pltpu.VMEM((EP, D // 2), jnp.int32)
    run = pl.kernel(
        _body,
        out_type=jax.ShapeDtypeStruct((B,), jnp.float32),
        mesh=mesh,
        compiler_params=pltpu.CompilerParams(needs_layout_passes=False,
                                             use_tc_tiling_on_sc=False),
        scratch_types=[
            pltpu.VMEM((CR, D), jnp.float32),
            pltpu.VMEM((CR, D // 2), jnp.int32),
            pltpu.VMEM((CR, D), jnp.float32),
            pltpu.VMEM((CR, D // 2), jnp.int32),
            pltpu.VMEM_SHARED((V, D // 2), jnp.int32),
            pltpu.VMEM((EW,), jnp.int32),
            pltpu.VMEM((EW,), jnp.int32),
            rows_t, rows_t, rows_t, rows_t,
            pltpu.VMEM((EP,), jnp.float32),
            pltpu.VMEM((EP,), jnp.float32),
            pltpu.SemaphoreType.DMA,
            pltpu.SemaphoreType.DMA,
            pltpu.SemaphoreType.DMA,
            pltpu.SemaphoreType.DMA,
            pltpu.SemaphoreType.DMA,
            pltpu.SemaphoreType.DMA,
        ],
    )
    return run(eidx, embeddings)


# speedup vs baseline: 1.0621x; 1.0359x over previous
import jax
import jax.numpy as jnp
from jax import lax
from jax.experimental import pallas as pl
from jax.experimental.pallas import tpu as pltpu
from jax.experimental.pallas import tpu_sc as plsc

NC = 2
NS = 16
NW = NC * NS
B = 320000
V = 10000
D = 128
E = 200
PER_W = B // (NW * E)
EW = B // NW
GROUPS = (E + 15) // 16
EP = GROUPS * 16
DEPTH = 2
CR = 25
CCH = V // (CR * NS)


def _body(eidx_hbm, emb_hbm, out_hbm, ci0, co0, ci1, co1, spm,
          idx_s, idx_o, rs0, ro0, rs1, ro1, ov0, ov1,
          sem0, sem1, semi0, semi1, semo0, semo1):
    cid = lax.axis_index("c")
    sid = lax.axis_index("s")
    wid = sid * NC + cid
    ebase = wid * EW
    lane = lax.iota(jnp.int32, 16)
    perms = [lane ^ m for m in (8, 4, 2, 1)]
    bufs = [(rs0, ro0, sem0), (rs1, ro1, sem1)]

    pltpu.async_copy(eidx_hbm.at[0, pl.ds(ebase, EW)], idx_s, sem0)
    pltpu.async_copy(eidx_hbm.at[1, pl.ds(ebase, EW)], idx_o, sem1)

    rbase = sid * (CR * CCH)
    cbufs = [(ci0, co0, semo0), (ci1, co1, semo1)]

    def conv_in_start(k, par):
        ci, _, _ = cbufs[par]
        pltpu.async_copy(emb_hbm.at[pl.ds(rbase + k * CR, CR), :], ci,
                         semi0 if par == 0 else semi1)

    def conv_in_wait(k, par):
        ci, _, _ = cbufs[par]
        pltpu.make_async_copy(emb_hbm.at[pl.ds(rbase + k * CR, CR), :], ci,
                              semi0 if par == 0 else semi1).wait()

    def conv_out_wait(k, par):
        _, co, semo = cbufs[par]
        pltpu.make_async_copy(co, spm.at[pl.ds(rbase + k * CR, CR), :],
                              semo).wait()

    def convert(k, par, wait_out):
        ci, co, semo = cbufs[par]
        conv_in_wait(k, par)
        if wait_out:
            conv_out_wait(k - 2, par)
        for r in range(CR):
            for j in range(D // 32):
                a = ci[r, pl.ds(j * 32, 16)]
                b = ci[r, pl.ds(j * 32 + 16, 16)]
                packed = plsc.pack(a, b, format=plsc.PackFormat.INTERLEAVED)
                co[r, pl.ds(j * 16, 16)] = plsc.bitcast(packed, jnp.int32)
        pltpu.async_copy(co, spm.at[pl.ds(rbase + k * CR, CR), :], semo)

    conv_in_start(0, 0)
    conv_in_start(1, 1)

    def conv_pair(p, carry):
        k0 = 2 * p
        convert(k0, 0, wait_out=True)
        conv_in_start(k0 + 2, 0)
        convert(k0 + 1, 1, wait_out=True)
        conv_in_start(k0 + 3, 1)
        return carry

    convert(0, 0, wait_out=False)
    conv_in_start(2, 0)
    convert(1, 1, wait_out=False)
    conv_in_start(3, 1)
    lax.fori_loop(1, 11, conv_pair, 0)
    convert(22, 0, wait_out=True)
    conv_in_start(24, 0)
    convert(23, 1, wait_out=True)
    convert(24, 0, wait_out=True)
    conv_out_wait(23, 1)
    conv_out_wait(24, 0)
    plsc.subcore_barrier()

    pltpu.make_async_copy(eidx_hbm.at[0, pl.ds(ebase, EW)], idx_s,
                          sem0).wait()
    pltpu.make_async_copy(eidx_hbm.at[1, pl.ds(ebase, EW)], idx_o,
                          sem1).wait()

    def start(c, k):
        rows_s, rows_o, sem = bufs[k]
        pltpu.async_copy(spm.at[idx_s.at[pl.ds(c * E, E)]],
                         rows_s.at[pl.ds(0, E)], sem)
        pltpu.async_copy(spm.at[idx_o.at[pl.ds(c * E, E)]],
                         rows_o.at[pl.ds(0, E)], sem)

    def finish(c, k):
        rows_s, rows_o, sem = bufs[k]
        pltpu.make_async_copy(spm.at[idx_s.at[pl.ds(c * E, E)]],
                              rows_s.at[pl.ds(0, E)], sem).wait()
        pltpu.make_async_copy(spm.at[idx_o.at[pl.ds(c * E, E)]],
                              rows_o.at[pl.ds(0, E)], sem).wait()

    ovs = [(ov0, semo0), (ov1, semo1)]

    def compute(c, k):
        rows_s, rows_o, _ = bufs[k]
        out_v, _ = ovs[k]

        def group(g, carry2):
            outv = jnp.zeros((16,), jnp.float32)
            for l in range(16):
                e = g * 16 + l
                ps = []
                for j in range(4):
                    sv = plsc.bitcast(rows_s[e, pl.ds(j * 16, 16)],
                                      jnp.bfloat16)
                    ov = plsc.bitcast(rows_o[e, pl.ds(j * 16, 16)],
                                      jnp.bfloat16)
                    ps.append(sv * ov)
                q0 = ps[0] + ps[1]
                q1 = ps[2] + ps[3]
                l0, h0 = plsc.unpack(q0, format=plsc.PackFormat.INTERLEAVED)
                l1, h1 = plsc.unpack(q1, format=plsc.PackFormat.INTERLEAVED)
                acc = (l0 + h0) + (l1 + h1)
                for pm in perms:
                    acc = acc + acc.at[pm].get(mode="promise_in_bounds",
                                               unique_indices=True)
                outv = jnp.where(lane == l, acc, outv)
            out_v[pl.ds(g * 16, 16)] = outv
            return carry2

        lax.fori_loop(0, GROUPS, group, 0)

    def store_start(c, k):
        out_v, semo = ovs[k]
        pltpu.async_copy(out_v.at[pl.ds(0, E)],
                         out_hbm.at[pl.ds(ebase + c * E, E)], semo)

    def store_wait(c, k):
        out_v, semo = ovs[k]
        pltpu.make_async_copy(out_v.at[pl.ds(0, E)],
                              out_hbm.at[pl.ds(ebase + c * E, E)],
                              semo).wait()

    start(0, 0)
    start(1, 1)
    for k in range(2):
        finish(k, k)
        compute(k, k)
        store_start(k, k)
        start(k + 2, k)

    def step(i, carry):
        c0 = DEPTH * i
        for k in range(DEPTH):
            finish(c0 + k, k)
            store_wait(c0 + k - 2, k)
            compute(c0 + k, k)
            store_start(c0 + k, k)
            start(c0 + k + 2, k)
        return carry

    lax.fori_loop(1, (PER_W - 2) // DEPTH, step, 0)

    for c in (PER_W - 2, PER_W - 1):
        finish(c, c % DEPTH)
        store_wait(c - 2, c % DEPTH)
        compute(c, c % DEPTH)
        store_start(c, c % DEPTH)
    store_wait(PER_W - 2, 0)
    store_wait(PER_W - 1, 1)


def kernel(embeddings, edge_index):
    eidx = edge_index.astype(jnp.int32)
    mesh = plsc.VectorSubcoreMesh(core_axis_name="c", subcore_axis_name="s")
    rows_t = pltpu.VMEM((EP, D // 2), jnp.int32)
    run = pl.kernel(
        _body,
        out_type=jax.ShapeDtypeStruct((B,), jnp.float32),
        mesh=mesh,
        compiler_params=pltpu.CompilerParams(needs_layout_passes=False,
                                             use_tc_tiling_on_sc=False),
        scratch_types=[
            pltpu.VMEM((CR, D), jnp.float32),
            pltpu.VMEM((CR, D // 2), jnp.int32),
            pltpu.VMEM((CR, D), jnp.float32),
            pltpu.VMEM((CR, D // 2), jnp.int32),
            pltpu.VMEM_SHARED((V, D // 2), jnp.int32),
            pltpu.VMEM((EW,), jnp.int32),
            pltpu.VMEM((EW,), jnp.int32),
            rows_t, rows_t, rows_t, rows_t,
            pltpu.VMEM((EP,), jnp.float32),
            pltpu.VMEM((EP,), jnp.float32),
            pltpu.SemaphoreType.DMA,
            pltpu.SemaphoreType.DMA,
            pltpu.SemaphoreType.DMA,
            pltpu.SemaphoreType.DMA,
            pltpu.SemaphoreType.DMA,
            pltpu.SemaphoreType.DMA,
        ],
    )
    return run(eidx, embeddings)


# speedup vs baseline: 1.0624x; 1.0002x over previous
import jax
import jax.numpy as jnp
from jax import lax
from jax.experimental import pallas as pl
from jax.experimental.pallas import tpu as pltpu
from jax.experimental.pallas import tpu_sc as plsc

NC = 2
NS = 16
NW = NC * NS
B = 320000
V = 10000
D = 128
E = 200
PER_W = B // (NW * E)
EW = B // NW
GROUPS = (E + 15) // 16
EP = GROUPS * 16
DEPTH = 2
CR = 25
CCH = V // (CR * NS)


def _body(eidx_hbm, emb_hbm, out_hbm, ci0, co0, ci1, co1, spm,
          idx_s, idx_o, rs0, ro0, rs1, ro1, ov0, ov1,
          sem0, sem1, semi0, semi1, semo0, semo1):
    cid = lax.axis_index("c")
    sid = lax.axis_index("s")
    wid = sid * NC + cid
    ebase = wid * EW
    lane = lax.iota(jnp.int32, 16)
    perms = [lane ^ m for m in (8, 4, 2, 1)]
    bufs = [(rs0, ro0, sem0), (rs1, ro1, sem1)]

    pltpu.async_copy(eidx_hbm.at[pl.ds(ebase, EW)], idx_s, sem0)
    pltpu.async_copy(eidx_hbm.at[pl.ds(B + ebase, EW)], idx_o, sem1)

    rbase = sid * (CR * CCH)
    cbufs = [(ci0, co0, semo0), (ci1, co1, semo1)]

    def conv_in_start(k, par):
        ci, _, _ = cbufs[par]
        pltpu.async_copy(emb_hbm.at[pl.ds(rbase + k * CR, CR), :], ci,
                         semi0 if par == 0 else semi1)

    def conv_in_wait(k, par):
        ci, _, _ = cbufs[par]
        pltpu.make_async_copy(emb_hbm.at[pl.ds(rbase + k * CR, CR), :], ci,
                              semi0 if par == 0 else semi1).wait()

    def conv_out_wait(k, par):
        _, co, semo = cbufs[par]
        pltpu.make_async_copy(co, spm.at[pl.ds(rbase + k * CR, CR), :],
                              semo).wait()

    def convert(k, par, wait_out):
        ci, co, semo = cbufs[par]
        conv_in_wait(k, par)
        if wait_out:
            conv_out_wait(k - 2, par)
        for r in range(CR):
            for j in range(D // 32):
                a = ci[r, pl.ds(j * 32, 16)]
                b = ci[r, pl.ds(j * 32 + 16, 16)]
                packed = plsc.pack(a, b, format=plsc.PackFormat.INTERLEAVED)
                co[r, pl.ds(j * 16, 16)] = plsc.bitcast(packed, jnp.int32)
        pltpu.async_copy(co, spm.at[pl.ds(rbase + k * CR, CR), :], semo)

    conv_in_start(0, 0)
    conv_in_start(1, 1)

    def conv_pair(p, carry):
        k0 = 2 * p
        convert(k0, 0, wait_out=True)
        conv_in_start(k0 + 2, 0)
        convert(k0 + 1, 1, wait_out=True)
        conv_in_start(k0 + 3, 1)
        return carry

    convert(0, 0, wait_out=False)
    conv_in_start(2, 0)
    convert(1, 1, wait_out=False)
    conv_in_start(3, 1)
    lax.fori_loop(1, 11, conv_pair, 0)
    convert(22, 0, wait_out=True)
    conv_in_start(24, 0)
    convert(23, 1, wait_out=True)
    convert(24, 0, wait_out=True)
    conv_out_wait(23, 1)
    conv_out_wait(24, 0)
    plsc.subcore_barrier()

    pltpu.make_async_copy(eidx_hbm.at[pl.ds(ebase, EW)], idx_s,
                          sem0).wait()
    pltpu.make_async_copy(eidx_hbm.at[pl.ds(B + ebase, EW)], idx_o,
                          sem1).wait()

    def start(c, k):
        rows_s, rows_o, sem = bufs[k]
        pltpu.async_copy(spm.at[idx_s.at[pl.ds(c * E, E)]],
                         rows_s.at[pl.ds(0, E)], sem)
        pltpu.async_copy(spm.at[idx_o.at[pl.ds(c * E, E)]],
                         rows_o.at[pl.ds(0, E)], sem)

    def finish(c, k):
        rows_s, rows_o, sem = bufs[k]
        pltpu.make_async_copy(spm.at[idx_s.at[pl.ds(c * E, E)]],
                              rows_s.at[pl.ds(0, E)], sem).wait()
        pltpu.make_async_copy(spm.at[idx_o.at[pl.ds(c * E, E)]],
                              rows_o.at[pl.ds(0, E)], sem).wait()

    ovs = [(ov0, semo0), (ov1, semo1)]

    def compute(c, k):
        rows_s, rows_o, _ = bufs[k]
        out_v, _ = ovs[k]

        def group(g, carry2):
            outv = jnp.zeros((16,), jnp.float32)
            for l in range(16):
                e = g * 16 + l
                ps = []
                for j in range(4):
                    sv = plsc.bitcast(rows_s[e, pl.ds(j * 16, 16)],
                                      jnp.bfloat16)
                    ov = plsc.bitcast(rows_o[e, pl.ds(j * 16, 16)],
                                      jnp.bfloat16)
                    ps.append(sv * ov)
                q0 = ps[0] + ps[1]
                q1 = ps[2] + ps[3]
                l0, h0 = plsc.unpack(q0, format=plsc.PackFormat.INTERLEAVED)
                l1, h1 = plsc.unpack(q1, format=plsc.PackFormat.INTERLEAVED)
                acc = (l0 + h0) + (l1 + h1)
                for pm in perms:
                    acc = acc + acc.at[pm].get(mode="promise_in_bounds",
                                               unique_indices=True)
                outv = jnp.where(lane == l, acc, outv)
            out_v[pl.ds(g * 16, 16)] = outv
            return carry2

        lax.fori_loop(0, GROUPS, group, 0)

    def store_start(c, k):
        out_v, semo = ovs[k]
        pltpu.async_copy(out_v.at[pl.ds(0, E)],
                         out_hbm.at[pl.ds(ebase + c * E, E)], semo)

    def store_wait(c, k):
        out_v, semo = ovs[k]
        pltpu.make_async_copy(out_v.at[pl.ds(0, E)],
                              out_hbm.at[pl.ds(ebase + c * E, E)],
                              semo).wait()

    start(0, 0)
    start(1, 1)
    for k in range(2):
        finish(k, k)
        compute(k, k)
        store_start(k, k)
        start(k + 2, k)

    def step(i, carry):
        c0 = DEPTH * i
        for k in range(DEPTH):
            finish(c0 + k, k)
            store_wait(c0 + k - 2, k)
            compute(c0 + k, k)
            store_start(c0 + k, k)
            start(c0 + k + 2, k)
        return carry

    lax.fori_loop(1, (PER_W - 2) // DEPTH, step, 0)

    for c in (PER_W - 2, PER_W - 1):
        finish(c, c % DEPTH)
        store_wait(c - 2, c % DEPTH)
        compute(c, c % DEPTH)
        store_start(c, c % DEPTH)
    store_wait(PER_W - 2, 0)
    store_wait(PER_W - 1, 1)


def kernel(embeddings, edge_index):
    eidx = edge_index.astype(jnp.int32).reshape(-1)
    mesh = plsc.VectorSubcoreMesh(core_axis_name="c", subcore_axis_name="s")
    rows_t = pltpu.VMEM((EP, D // 2), jnp.int32)
    run = pl.kernel(
        _body,
        out_type=jax.ShapeDtypeStruct((B,), jnp.float32),
        mesh=mesh,
        compiler_params=pltpu.CompilerParams(needs_layout_passes=False,
                                             use_tc_tiling_on_sc=False),
        scratch_types=[
            pltpu.VMEM((CR, D), jnp.float32),
            pltpu.VMEM((CR, D // 2), jnp.int32),
            pltpu.VMEM((CR, D), jnp.float32),
            pltpu.VMEM((CR, D // 2), jnp.int32),
            pltpu.VMEM_SHARED((V, D // 2), jnp.int32),
            pltpu.VMEM((EW,), jnp.int32),
            pltpu.VMEM((EW,), jnp.int32),
            rows_t, rows_t, rows_t, rows_t,
            pltpu.VMEM((EP,), jnp.float32),
            pltpu.VMEM((EP,), jnp.float32),
            pltpu.SemaphoreType.DMA,
            pltpu.SemaphoreType.DMA,
            pltpu.SemaphoreType.DMA,
            pltpu.SemaphoreType.DMA,
            pltpu.SemaphoreType.DMA,
            pltpu.SemaphoreType.DMA,
        ],
    )
    return run(eidx, embeddings)
